# Initial kernel scaffold; baseline (speedup 1.0000x reference)
import jax
import jax.numpy as jnp
from jax.experimental import pallas as pl


def kernel(local_ids, U, B):
    raise NotImplementedError("write your pallas kernel here")


# speedup vs baseline: 8.0422x; 8.0422x over previous
import functools

import jax
import jax.numpy as jnp
from jax import lax
from jax.experimental import pallas as pl
from jax.experimental.pallas import tpu as pltpu
from jax.experimental.pallas import tpu_sc as plsc

_R = 64
_D = 128

_NC = 2
_NS = 16
_NW = _NC * _NS

_CH = 128


def _matmul_body(u_ref, b_ref, w_ref):
    w_ref[...] = jnp.dot(u_ref[...], b_ref[...],
                         preferred_element_type=jnp.float32)


def _compute_w(U, B):
    m = U.shape[0]
    blk = 2000
    return pl.pallas_call(
        _matmul_body,
        grid=(m // blk,),
        in_specs=[
            pl.BlockSpec((blk, _R), lambda i: (i, 0)),
            pl.BlockSpec((_R, _D), lambda i: (0, 0)),
        ],
        out_specs=pl.BlockSpec((blk, _D), lambda i: (i, 0)),
        out_shape=jax.ShapeDtypeStruct((m, _D), jnp.float32),
    )(U, B)


def _make_gather(ntot, nch):
    mesh = plsc.VectorSubcoreMesh(core_axis_name="c", subcore_axis_name="s")

    @functools.partial(
        pl.kernel,
        out_type=jax.ShapeDtypeStruct((ntot, _D), jnp.float32),
        mesh=mesh,
        scratch_types=[
            pltpu.VMEM((nch, _CH), jnp.int32),
            pltpu.VMEM((_CH, _D), jnp.float32),
            pltpu.VMEM((_CH, _D), jnp.float32),
            pltpu.SemaphoreType.DMA,
            pltpu.SemaphoreType.DMA,
        ],
    )
    def gather(table_hbm, idx_hbm, out_hbm, idx_v, rows0, rows1, sem0, sem1):
        wid = lax.axis_index("s") * _NC + lax.axis_index("c")
        base = wid * (nch * _CH)
        pltpu.sync_copy(idx_hbm.at[wid], idx_v)

        rows = (rows0, rows1)
        sems = (sem0, sem1)

        pltpu.make_async_copy(table_hbm.at[idx_v.at[0]], rows0, sem0).start()

        def body(g, _):
            for b in range(2):
                c = 2 * g + b
                nxt = c + 1

                @pl.when(nxt < nch)
                def _():
                    pltpu.make_async_copy(
                        table_hbm.at[idx_v.at[nxt]], rows[1 - b], sems[1 - b]
                    ).start()

                pltpu.make_async_copy(
                    table_hbm.at[idx_v.at[c]], rows[b], sems[b]
                ).wait()
                pltpu.sync_copy(
                    rows[b], out_hbm.at[pl.ds(base + c * _CH, _CH)]
                )
            return 0

        lax.fori_loop(0, nch // 2, body, 0, unroll=False)

    return gather


def kernel(local_ids, U, B):
    bsz, seq = local_ids.shape
    ntot = bsz * seq
    nch = ntot // (_NW * _CH)

    W = _compute_w(U, B)
    ids3 = local_ids.astype(jnp.int32).reshape(_NW, nch, _CH)
    out = _make_gather(ntot, nch)(W, ids3)
    return out.reshape(bsz, seq, _D)
